# Initial kernel scaffold; baseline (speedup 1.0000x reference)
import functools

import jax
import jax.numpy as jnp
from jax import lax
from jax.experimental import pallas as pl
from jax.experimental.pallas import tpu as pltpu
from jax.experimental.pallas import tpu_sc as plsc

N = 10000
NP = 10240
E = 320000
D = 128
H = 16
C = 64

NW = 32
EW = E // NW
CH = 80
NCH = EW // CH
RPS = NP // 16

_MESH = dict(core_axis_name="c", subcore_axis_name="s")


def _sc_mesh():
    return plsc.VectorSubcoreMesh(**_MESH)


@functools.partial(
    pl.kernel,
    mesh=_sc_mesh(),
    out_type=jax.ShapeDtypeStruct((2, NP), jnp.float32),
    scratch_types=[
        pltpu.VMEM_SHARED((NP,), jnp.float32),
        pltpu.VMEM((CH,), jnp.int32),
        pltpu.VMEM((CH,), jnp.float32),
        pltpu.VMEM((RPS,), jnp.float32),
    ],
)
def _deg_sc(dst_hbm, out_hbm, deg_sp, idx_v, ones_v, zero_v):
    cid = lax.axis_index("c")
    sid = lax.axis_index("s")
    for i in range(RPS // 16):
        zero_v[pl.ds(i * 16, 16)] = jnp.zeros((16,), jnp.float32)
    for i in range(CH // 16):
        ones_v[pl.ds(i * 16, 16)] = jnp.full((16,), 1.0, jnp.float32)
    pltpu.sync_copy(zero_v, deg_sp.at[pl.ds(sid * RPS, RPS)])
    plsc.subcore_barrier()
    base = (cid * 16 + sid) * EW

    def body(j, carry):
        pltpu.sync_copy(dst_hbm.at[pl.ds(base + j * CH, CH)], idx_v)
        pltpu.sync_copy(ones_v, deg_sp.at[idx_v], add=True)
        return carry

    lax.fori_loop(0, NCH, body, 0)
    plsc.subcore_barrier()
    pltpu.sync_copy(deg_sp.at[pl.ds(sid * RPS, RPS)],
                    out_hbm.at[cid, pl.ds(sid * RPS, RPS)])


@functools.partial(
    pl.kernel,
    mesh=_sc_mesh(),
    out_type=jax.ShapeDtypeStruct((2, NP, H), jnp.float32),
    scratch_types=[
        pltpu.VMEM_SHARED((NP, H), jnp.float32),
        pltpu.VMEM((CH,), jnp.int32),
        pltpu.VMEM((CH,), jnp.int32),
        pltpu.VMEM((CH, H), jnp.float32),
        pltpu.SemaphoreType.DMA,
    ],
)
def _edge_scatter_sc(t_hbm, src_hbm, dst_hbm, out_hbm,
                     acc_sp, sidx_v, didx_v, rows_v, sem):
    cid = lax.axis_index("c")
    sid = lax.axis_index("s")
    pltpu.sync_copy(t_hbm.at[pl.ds(sid * RPS, RPS)],
                    acc_sp.at[pl.ds(sid * RPS, RPS)])
    plsc.subcore_barrier()
    base = (cid * 16 + sid) * EW

    def body(j, carry):
        pltpu.sync_copy(src_hbm.at[pl.ds(base + j * CH, CH)], sidx_v)
        pltpu.sync_copy(dst_hbm.at[pl.ds(base + j * CH, CH)], didx_v)
        pltpu.async_copy(t_hbm.at[sidx_v], rows_v, sem).wait()
        pltpu.sync_copy(rows_v, acc_sp.at[didx_v], add=True)
        return carry

    lax.fori_loop(0, NCH, body, 0)
    plsc.subcore_barrier()
    pltpu.sync_copy(acc_sp.at[pl.ds(sid * RPS, RPS)],
                    out_hbm.at[cid, pl.ds(sid * RPS, RPS)])


def _tc_in_body(x_ref, w_ref, dsum_ref, t_ref):
    dinv = lax.rsqrt(dsum_ref[...])
    h = jnp.dot(x_ref[...], w_ref[...], preferred_element_type=jnp.float32)
    t_ref[...] = h * dinv


def _tc_mid_body(s_ref, t1_ref, dsum_ref, b_ref, t2_ref):
    dinv = lax.rsqrt(dsum_ref[...])
    pre = dinv * (s_ref[0] + s_ref[1] - t1_ref[...]) + b_ref[...]
    t2_ref[...] = dinv * jnp.maximum(pre, 0.0)


def _tc_out_body(s_ref, t2_ref, dsum_ref, w_ref, b_ref, o_ref):
    dinv = lax.rsqrt(dsum_ref[...])
    agg = dinv * (s_ref[0] + s_ref[1] - t2_ref[...])
    z = jnp.dot(agg, w_ref[...], preferred_element_type=jnp.float32) + b_ref[...]
    m = jnp.max(z, axis=1, keepdims=True)
    lse = m + jnp.log(jnp.sum(jnp.exp(z - m), axis=1, keepdims=True))
    o_ref[...] = z - lse


_tc_in = pl.pallas_call(
    _tc_in_body, out_shape=jax.ShapeDtypeStruct((NP, H), jnp.float32))
_tc_mid = pl.pallas_call(
    _tc_mid_body, out_shape=jax.ShapeDtypeStruct((NP, H), jnp.float32))
_tc_out = pl.pallas_call(
    _tc_out_body, out_shape=jax.ShapeDtypeStruct((NP, C), jnp.float32))


def kernel(x, edge_index, W1, b1, W2, b2):
    ei = edge_index.astype(jnp.int32)
    src, dst = ei[0], ei[1]
    x_pad = jnp.pad(x, ((0, NP - N), (0, 0)))

    degs = _deg_sc(dst)
    dsum = (degs[0] + degs[1] + 1.0).reshape(NP, 1)

    t1 = _tc_in(x_pad, W1, dsum)
    s1 = _edge_scatter_sc(t1, src, dst)
    t2 = _tc_mid(s1, t1, dsum, b1.reshape(1, H))
    s2 = _edge_scatter_sc(t2, src, dst)
    out = _tc_out(s2, t2, dsum, W2, b2.reshape(1, C))
    return out[:N]


# speedup vs baseline: 16.8190x; 16.8190x over previous
import functools

import jax
import jax.numpy as jnp
from jax import lax
from jax.experimental import pallas as pl
from jax.experimental.pallas import tpu as pltpu
from jax.experimental.pallas import tpu_sc as plsc

N = 10000
NP = 10240
E = 320000
D = 128
H = 16
C = 64

NW = 32
EW = E // NW
CH = 80
NCH = EW // CH
RPS = NP // 16

_MESH = dict(core_axis_name="c", subcore_axis_name="s")


def _sc_mesh():
    return plsc.VectorSubcoreMesh(**_MESH)


_SC_PARAMS = pltpu.CompilerParams(use_tc_tiling_on_sc=False)


@functools.partial(
    pl.kernel,
    mesh=_sc_mesh(),
    compiler_params=_SC_PARAMS,
    out_type=jax.ShapeDtypeStruct((2, NP), jnp.float32),
    scratch_types=[
        pltpu.VMEM_SHARED((NP,), jnp.float32),
        pltpu.VMEM((CH,), jnp.int32),
        pltpu.VMEM((CH,), jnp.float32),
        pltpu.VMEM((RPS,), jnp.float32),
    ],
)
def _deg_sc(dst_hbm, out_hbm, deg_sp, idx_v, ones_v, zero_v):
    cid = lax.axis_index("c")
    sid = lax.axis_index("s")
    for i in range(RPS // 16):
        zero_v[pl.ds(i * 16, 16)] = jnp.zeros((16,), jnp.float32)
    for i in range(CH // 16):
        ones_v[pl.ds(i * 16, 16)] = jnp.full((16,), 1.0, jnp.float32)
    pltpu.sync_copy(zero_v, deg_sp.at[pl.ds(sid * RPS, RPS)])
    plsc.subcore_barrier()
    base = (cid * 16 + sid) * EW

    def body(j, carry):
        pltpu.sync_copy(dst_hbm.at[pl.ds(base + j * CH, CH)], idx_v)
        pltpu.sync_copy(ones_v, deg_sp.at[idx_v], add=True)
        return carry

    lax.fori_loop(0, NCH, body, 0)
    plsc.subcore_barrier()
    pltpu.sync_copy(deg_sp.at[pl.ds(sid * RPS, RPS)],
                    out_hbm.at[cid, pl.ds(sid * RPS, RPS)])


@functools.partial(
    pl.kernel,
    mesh=_sc_mesh(),
    compiler_params=_SC_PARAMS,
    out_type=jax.ShapeDtypeStruct((2, NP, H), jnp.float32),
    scratch_types=[
        pltpu.VMEM_SHARED((NP, H), jnp.float32),
        pltpu.VMEM((CH,), jnp.int32),
        pltpu.VMEM((CH,), jnp.int32),
        pltpu.VMEM((CH, H), jnp.float32),
        pltpu.SemaphoreType.DMA,
    ],
)
def _edge_scatter_sc(t_hbm, src_hbm, dst_hbm, out_hbm,
                     acc_sp, sidx_v, didx_v, rows_v, sem):
    cid = lax.axis_index("c")
    sid = lax.axis_index("s")
    pltpu.sync_copy(t_hbm.at[pl.ds(sid * RPS, RPS)],
                    acc_sp.at[pl.ds(sid * RPS, RPS)])
    plsc.subcore_barrier()
    base = (cid * 16 + sid) * EW

    def body(j, carry):
        pltpu.sync_copy(src_hbm.at[pl.ds(base + j * CH, CH)], sidx_v)
        pltpu.sync_copy(dst_hbm.at[pl.ds(base + j * CH, CH)], didx_v)
        pltpu.async_copy(t_hbm.at[sidx_v], rows_v, sem).wait()
        pltpu.sync_copy(rows_v, acc_sp.at[didx_v], add=True)
        return carry

    lax.fori_loop(0, NCH, body, 0)
    plsc.subcore_barrier()
    pltpu.sync_copy(acc_sp.at[pl.ds(sid * RPS, RPS)],
                    out_hbm.at[cid, pl.ds(sid * RPS, RPS)])


def _tc_in_body(x_ref, w_ref, dsum_ref, t_ref):
    dinv = lax.rsqrt(dsum_ref[...])
    h = jnp.dot(x_ref[...], w_ref[...], preferred_element_type=jnp.float32)
    t_ref[...] = h * dinv


def _tc_mid_body(s_ref, t1_ref, dsum_ref, b_ref, t2_ref):
    dinv = lax.rsqrt(dsum_ref[...])
    pre = dinv * (s_ref[0] + s_ref[1] - t1_ref[...]) + b_ref[...]
    t2_ref[...] = dinv * jnp.maximum(pre, 0.0)


def _tc_out_body(s_ref, t2_ref, dsum_ref, w_ref, b_ref, o_ref):
    dinv = lax.rsqrt(dsum_ref[...])
    agg = dinv * (s_ref[0] + s_ref[1] - t2_ref[...])
    z = jnp.dot(agg, w_ref[...], preferred_element_type=jnp.float32) + b_ref[...]
    m = jnp.max(z, axis=1, keepdims=True)
    lse = m + jnp.log(jnp.sum(jnp.exp(z - m), axis=1, keepdims=True))
    o_ref[...] = z - lse


_tc_in = pl.pallas_call(
    _tc_in_body, out_shape=jax.ShapeDtypeStruct((NP, H), jnp.float32))
_tc_mid = pl.pallas_call(
    _tc_mid_body, out_shape=jax.ShapeDtypeStruct((NP, H), jnp.float32))
_tc_out = pl.pallas_call(
    _tc_out_body, out_shape=jax.ShapeDtypeStruct((NP, C), jnp.float32))


def kernel(x, edge_index, W1, b1, W2, b2):
    ei = edge_index.astype(jnp.int32)
    src, dst = ei[0], ei[1]
    x_pad = jnp.pad(x, ((0, NP - N), (0, 0)))

    degs = _deg_sc(dst)
    dsum = (degs[0] + degs[1] + 1.0).reshape(NP, 1)

    t1 = _tc_in(x_pad, W1, dsum)
    s1 = _edge_scatter_sc(t1, src, dst)
    t2 = _tc_mid(s1, t1, dsum, b1.reshape(1, H))
    s2 = _edge_scatter_sc(t2, src, dst)
    out = _tc_out(s2, t2, dsum, W2, b2.reshape(1, C))
    return out[:N]


# speedup vs baseline: 45.0640x; 2.6793x over previous
import functools

import jax
import jax.numpy as jnp
from jax import lax
from jax.experimental import pallas as pl
from jax.experimental.pallas import tpu as pltpu
from jax.experimental.pallas import tpu_sc as plsc

N = 10000
NP = 10240
E = 320000
D = 128
H = 16
C = 64

NW = 32
EW = E // NW
CH = 80
NCH = EW // CH
RPS = NP // 16

_MESH = dict(core_axis_name="c", subcore_axis_name="s")


def _sc_mesh():
    return plsc.VectorSubcoreMesh(**_MESH)


_SC_PARAMS = pltpu.CompilerParams(use_tc_tiling_on_sc=False)


@functools.partial(
    pl.kernel,
    mesh=_sc_mesh(),
    compiler_params=_SC_PARAMS,
    out_type=jax.ShapeDtypeStruct((2, NP), jnp.float32),
    scratch_types=[
        pltpu.VMEM_SHARED((NP,), jnp.float32),
        pltpu.VMEM((NCH, CH), jnp.int32),
        pltpu.VMEM((CH,), jnp.float32),
        pltpu.VMEM((RPS,), jnp.float32),
        pltpu.SemaphoreType.DMA,
        pltpu.SemaphoreType.DMA,
    ],
)
def _deg_sc(dst_hbm, out_hbm, deg_sp, didx_v, ones_v, zero_v, isem, ssem):
    cid = lax.axis_index("c")
    sid = lax.axis_index("s")
    wid = cid * 16 + sid
    ld = pltpu.async_copy(dst_hbm.at[wid], didx_v, isem)
    for i in range(RPS // 16):
        zero_v[pl.ds(i * 16, 16)] = jnp.zeros((16,), jnp.float32)
    for i in range(CH // 16):
        ones_v[pl.ds(i * 16, 16)] = jnp.full((16,), 1.0, jnp.float32)
    pltpu.sync_copy(zero_v, deg_sp.at[pl.ds(sid * RPS, RPS)])
    ld.wait()
    plsc.subcore_barrier()

    def s_start(j):
        pltpu.async_copy(ones_v, deg_sp.at[didx_v.at[j]], ssem, add=True)

    def s_wait(j):
        pltpu.make_async_copy(ones_v, deg_sp.at[didx_v.at[j]], ssem).wait()

    for j in range(4):
        s_start(j)

    def body(j, carry):
        s_start(j)
        s_wait(j - 4)
        return carry

    lax.fori_loop(4, NCH, body, 0)
    for j in range(NCH - 4, NCH):
        s_wait(j)
    plsc.subcore_barrier()
    pltpu.sync_copy(deg_sp.at[pl.ds(sid * RPS, RPS)],
                    out_hbm.at[cid, pl.ds(sid * RPS, RPS)])


@functools.partial(
    pl.kernel,
    mesh=_sc_mesh(),
    compiler_params=_SC_PARAMS,
    out_type=jax.ShapeDtypeStruct((2, NP, H), jnp.float32),
    scratch_types=[
        pltpu.VMEM_SHARED((NP, H), jnp.float32),
        pltpu.VMEM((NCH, CH), jnp.int32),
        pltpu.VMEM((NCH, CH), jnp.int32),
        pltpu.VMEM((2, CH, H), jnp.float32),
        pltpu.SemaphoreType.DMA((2,)),
        pltpu.SemaphoreType.DMA((2,)),
        pltpu.SemaphoreType.DMA,
    ],
)
def _edge_scatter_sc(t_hbm, src_hbm, dst_hbm, out_hbm,
                     acc_sp, sidx_v, didx_v, rows_v, gsem, ssem, isem):
    cid = lax.axis_index("c")
    sid = lax.axis_index("s")
    wid = cid * 16 + sid
    l1 = pltpu.async_copy(src_hbm.at[wid], sidx_v, isem)
    l2 = pltpu.async_copy(dst_hbm.at[wid], didx_v, isem)
    l3 = pltpu.async_copy(t_hbm.at[pl.ds(sid * RPS, RPS)],
                          acc_sp.at[pl.ds(sid * RPS, RPS)], isem)
    l1.wait()
    l2.wait()
    l3.wait()
    plsc.subcore_barrier()

    def g_start(j, b):
        pltpu.async_copy(t_hbm.at[sidx_v.at[j]], rows_v.at[b], gsem.at[b])

    def g_wait(j, b):
        pltpu.make_async_copy(t_hbm.at[sidx_v.at[j]], rows_v.at[b],
                              gsem.at[b]).wait()

    def s_start(j, b):
        pltpu.async_copy(rows_v.at[b], acc_sp.at[didx_v.at[j]], ssem.at[b],
                         add=True)

    def s_wait(j, b):
        pltpu.make_async_copy(rows_v.at[b], acc_sp.at[didx_v.at[j]],
                              ssem.at[b]).wait()

    g_start(0, 0)
    g_start(1, 1)
    g_wait(0, 0)
    s_start(0, 0)

    def body(j, carry):
        b = j % 2
        nb = 1 - b
        s_wait(j - 1, nb)
        g_start(j + 1, nb)
        g_wait(j, b)
        s_start(j, b)
        return carry

    lax.fori_loop(1, NCH - 1, body, 0)
    jl = NCH - 1
    bl = jl % 2
    s_wait(jl - 1, 1 - bl)
    g_wait(jl, bl)
    s_start(jl, bl)
    s_wait(jl, bl)
    plsc.subcore_barrier()
    pltpu.sync_copy(acc_sp.at[pl.ds(sid * RPS, RPS)],
                    out_hbm.at[cid, pl.ds(sid * RPS, RPS)])


def _tc_in_body(x_ref, w_ref, dsum_ref, t_ref):
    dinv = lax.rsqrt(dsum_ref[...])
    h = jnp.dot(x_ref[...], w_ref[...], preferred_element_type=jnp.float32)
    t_ref[...] = h * dinv


def _tc_mid_body(s_ref, t1_ref, dsum_ref, b_ref, t2_ref):
    dinv = lax.rsqrt(dsum_ref[...])
    pre = dinv * (s_ref[0] + s_ref[1] - t1_ref[...]) + b_ref[...]
    t2_ref[...] = dinv * jnp.maximum(pre, 0.0)


def _tc_out_body(s_ref, t2_ref, dsum_ref, w_ref, b_ref, o_ref):
    dinv = lax.rsqrt(dsum_ref[...])
    agg = dinv * (s_ref[0] + s_ref[1] - t2_ref[...])
    z = jnp.dot(agg, w_ref[...], preferred_element_type=jnp.float32) + b_ref[...]
    m = jnp.max(z, axis=1, keepdims=True)
    lse = m + jnp.log(jnp.sum(jnp.exp(z - m), axis=1, keepdims=True))
    o_ref[...] = z - lse


_tc_in = pl.pallas_call(
    _tc_in_body, out_shape=jax.ShapeDtypeStruct((NP, H), jnp.float32))
_tc_mid = pl.pallas_call(
    _tc_mid_body, out_shape=jax.ShapeDtypeStruct((NP, H), jnp.float32))
_tc_out = pl.pallas_call(
    _tc_out_body, out_shape=jax.ShapeDtypeStruct((NP, C), jnp.float32))


def kernel(x, edge_index, W1, b1, W2, b2):
    ei = edge_index.astype(jnp.int32)
    src = ei[0].reshape(NW, NCH, CH)
    dst = ei[1].reshape(NW, NCH, CH)
    x_pad = jnp.pad(x, ((0, NP - N), (0, 0)))

    degs = _deg_sc(dst)
    dsum = (degs[0] + degs[1] + 1.0).reshape(NP, 1)

    t1 = _tc_in(x_pad, W1, dsum)
    s1 = _edge_scatter_sc(t1, src, dst)
    t2 = _tc_mid(s1, t1, dsum, b1.reshape(1, H))
    s2 = _edge_scatter_sc(t2, src, dst)
    out = _tc_out(s2, t2, dsum, W2, b2.reshape(1, C))
    return out[:N]


# speedup vs baseline: 62.9115x; 1.3960x over previous
import functools

import jax
import jax.numpy as jnp
from jax import lax
from jax.experimental import pallas as pl
from jax.experimental.pallas import tpu as pltpu
from jax.experimental.pallas import tpu_sc as plsc

N = 10000
NP = 10240
E = 320000
D = 128
H = 16
C = 64

NW = 32
EW = E // NW
CH = 80
NCH = EW // CH
RPS = NP // 16

_MESH = dict(core_axis_name="c", subcore_axis_name="s")


def _sc_mesh():
    return plsc.VectorSubcoreMesh(**_MESH)


_SC_PARAMS = pltpu.CompilerParams(use_tc_tiling_on_sc=False)


@functools.partial(
    pl.kernel,
    mesh=_sc_mesh(),
    compiler_params=_SC_PARAMS,
    out_type=jax.ShapeDtypeStruct((2, NP), jnp.float32),
    scratch_types=[
        pltpu.VMEM_SHARED((NP,), jnp.float32),
        pltpu.VMEM((NCH, CH), jnp.int32),
        pltpu.VMEM((CH,), jnp.float32),
        pltpu.VMEM((RPS,), jnp.float32),
        pltpu.SemaphoreType.DMA,
        pltpu.SemaphoreType.DMA,
    ],
)
def _deg_sc(dst_hbm, out_hbm, deg_sp, didx_v, ones_v, zero_v, isem, ssem):
    cid = lax.axis_index("c")
    sid = lax.axis_index("s")
    wid = cid * 16 + sid
    ld = pltpu.async_copy(dst_hbm.at[wid], didx_v, isem)
    for i in range(RPS // 16):
        zero_v[pl.ds(i * 16, 16)] = jnp.zeros((16,), jnp.float32)
    for i in range(CH // 16):
        ones_v[pl.ds(i * 16, 16)] = jnp.full((16,), 1.0, jnp.float32)
    pltpu.sync_copy(zero_v, deg_sp.at[pl.ds(sid * RPS, RPS)])
    ld.wait()
    plsc.subcore_barrier()

    def s_start(j):
        pltpu.async_copy(ones_v, deg_sp.at[didx_v.at[j]], ssem, add=True)

    def s_wait(j):
        pltpu.make_async_copy(ones_v, deg_sp.at[didx_v.at[j]], ssem).wait()

    for j in range(4):
        s_start(j)

    def body(j, carry):
        s_start(j)
        s_wait(j - 4)
        return carry

    lax.fori_loop(4, NCH, body, 0)
    for j in range(NCH - 4, NCH):
        s_wait(j)
    plsc.subcore_barrier()
    pltpu.sync_copy(deg_sp.at[pl.ds(sid * RPS, RPS)],
                    out_hbm.at[cid, pl.ds(sid * RPS, RPS)])


CHE = 128
NROW = E // CHE
MAXJ = 79


@functools.partial(
    pl.kernel,
    mesh=_sc_mesh(),
    compiler_params=_SC_PARAMS,
    out_type=jax.ShapeDtypeStruct((2, NP, H), jnp.float32),
    scratch_types=[
        pltpu.VMEM_SHARED((NP, H), jnp.float32),
        pltpu.VMEM((MAXJ, CHE), jnp.int32),
        pltpu.VMEM((MAXJ, CHE), jnp.int32),
        pltpu.VMEM((4, CHE, H), jnp.float32),
        pltpu.SemaphoreType.DMA((4,)),
        pltpu.SemaphoreType.DMA((4,)),
        pltpu.SemaphoreType.DMA,
    ],
)
def _edge_scatter_sc(t_hbm, src_hbm, dst_hbm, out_hbm,
                     acc_sp, sidx_v, didx_v, rows_v, gsem, ssem, isem):
    cid = lax.axis_index("c")
    sid = lax.axis_index("s")
    wid = cid * 16 + sid
    start = 78 * wid + jnp.maximum(wid - 28, 0)
    nj = 78 + (wid >= 28).astype(jnp.int32)
    l1 = pltpu.async_copy(src_hbm.at[pl.ds(start, MAXJ)], sidx_v, isem)
    l2 = pltpu.async_copy(dst_hbm.at[pl.ds(start, MAXJ)], didx_v, isem)
    l3 = pltpu.async_copy(t_hbm.at[pl.ds(sid * RPS, RPS)],
                          acc_sp.at[pl.ds(sid * RPS, RPS)], isem)
    l1.wait()
    l2.wait()
    l3.wait()
    plsc.subcore_barrier()

    def g_start(j, b):
        pltpu.async_copy(t_hbm.at[sidx_v.at[j]], rows_v.at[b], gsem.at[b])

    def g_wait(j, b):
        pltpu.make_async_copy(t_hbm.at[sidx_v.at[j]], rows_v.at[b],
                              gsem.at[b]).wait()

    def s_start(j, b):
        pltpu.async_copy(rows_v.at[b], acc_sp.at[didx_v.at[j]], ssem.at[b],
                         add=True)

    def s_wait(j, b):
        pltpu.make_async_copy(rows_v.at[b], acc_sp.at[didx_v.at[j]],
                              ssem.at[b]).wait()

    for j in range(4):
        g_start(j, j)
    g_wait(0, 0)
    s_start(0, 0)

    def body(j, carry):
        b = j % 4
        s_wait(j - 1, (j - 1) % 4)

        @pl.when(j + 3 < nj)
        def _():
            g_start(j + 3, (j + 3) % 4)

        g_wait(j, b)
        s_start(j, b)
        return carry

    lax.fori_loop(1, nj, body, 0)
    s_wait(nj - 1, (nj - 1) % 4)
    plsc.subcore_barrier()
    pltpu.sync_copy(acc_sp.at[pl.ds(sid * RPS, RPS)],
                    out_hbm.at[cid, pl.ds(sid * RPS, RPS)])


def _tc_in_body(x_ref, w_ref, dsum_ref, t_ref):
    dinv = lax.rsqrt(dsum_ref[...])
    h = jnp.dot(x_ref[...], w_ref[...], preferred_element_type=jnp.float32)
    t_ref[pl.ds(0, N), :] = h * dinv[:N]
    t_ref[pl.ds(N, NP - N), :] = jnp.zeros((NP - N, H), jnp.float32)


def _tc_mid_body(s_ref, t1_ref, dsum_ref, b_ref, t2_ref):
    dinv = lax.rsqrt(dsum_ref[...])
    pre = dinv * (s_ref[0] + s_ref[1] - t1_ref[...]) + b_ref[...]
    t2_ref[...] = dinv * jnp.maximum(pre, 0.0)


def _tc_out_body(s_ref, t2_ref, dsum_ref, w_ref, b_ref, o_ref):
    dinv = lax.rsqrt(dsum_ref[...])
    agg = dinv * (s_ref[0] + s_ref[1] - t2_ref[...])
    z = jnp.dot(agg[:N], w_ref[...], preferred_element_type=jnp.float32) + b_ref[...]
    m = jnp.max(z, axis=1, keepdims=True)
    lse = m + jnp.log(jnp.sum(jnp.exp(z - m), axis=1, keepdims=True))
    o_ref[...] = z - lse


_tc_in = pl.pallas_call(
    _tc_in_body, out_shape=jax.ShapeDtypeStruct((NP, H), jnp.float32))
_tc_mid = pl.pallas_call(
    _tc_mid_body, out_shape=jax.ShapeDtypeStruct((NP, H), jnp.float32))
_tc_out = pl.pallas_call(
    _tc_out_body, out_shape=jax.ShapeDtypeStruct((N, C), jnp.float32))


def kernel(x, edge_index, W1, b1, W2, b2):
    ei = edge_index.astype(jnp.int32)
    dst_deg = ei[1].reshape(NW, NCH, CH)
    src = ei[0].reshape(NROW, CHE)
    dst = ei[1].reshape(NROW, CHE)

    degs = _deg_sc(dst_deg)
    dsum = (degs[0] + degs[1] + 1.0).reshape(NP, 1)

    t1 = _tc_in(x, W1, dsum)
    s1 = _edge_scatter_sc(t1, src, dst)
    t2 = _tc_mid(s1, t1, dsum, b1.reshape(1, H))
    s2 = _edge_scatter_sc(t2, src, dst)
    return _tc_out(s2, t2, dsum, W2, b2.reshape(1, C))


# speedup vs baseline: 66.0783x; 1.0503x over previous
import functools

import jax
import jax.numpy as jnp
from jax import lax
from jax.experimental import pallas as pl
from jax.experimental.pallas import tpu as pltpu
from jax.experimental.pallas import tpu_sc as plsc

N = 10000
NP = 10240
E = 320000
D = 128
H = 16
C = 64

NW = 32
EW = E // NW
CH = 80
NCH = EW // CH
RPS = NP // 16

_MESH = dict(core_axis_name="c", subcore_axis_name="s")


def _sc_mesh():
    return plsc.VectorSubcoreMesh(**_MESH)


_SC_PARAMS = pltpu.CompilerParams(use_tc_tiling_on_sc=False)


@functools.partial(
    pl.kernel,
    mesh=_sc_mesh(),
    compiler_params=_SC_PARAMS,
    out_type=jax.ShapeDtypeStruct((2, NP), jnp.float32),
    scratch_types=[
        pltpu.VMEM_SHARED((NP,), jnp.float32),
        pltpu.VMEM((NCH, CH), jnp.int32),
        pltpu.VMEM((CH,), jnp.float32),
        pltpu.VMEM((RPS,), jnp.float32),
        pltpu.SemaphoreType.DMA,
        pltpu.SemaphoreType.DMA,
    ],
)
def _deg_sc(dst_hbm, out_hbm, deg_sp, didx_v, ones_v, zero_v, isem, ssem):
    cid = lax.axis_index("c")
    sid = lax.axis_index("s")
    wid = cid * 16 + sid
    ld = pltpu.async_copy(dst_hbm.at[wid], didx_v, isem)
    for i in range(RPS // 16):
        zero_v[pl.ds(i * 16, 16)] = jnp.zeros((16,), jnp.float32)
    for i in range(CH // 16):
        ones_v[pl.ds(i * 16, 16)] = jnp.full((16,), 1.0, jnp.float32)
    pltpu.sync_copy(zero_v, deg_sp.at[pl.ds(sid * RPS, RPS)])
    ld.wait()
    plsc.subcore_barrier()

    def s_start(j):
        pltpu.async_copy(ones_v, deg_sp.at[didx_v.at[j]], ssem, add=True)

    def s_wait(j):
        pltpu.make_async_copy(ones_v, deg_sp.at[didx_v.at[j]], ssem).wait()

    for j in range(4):
        s_start(j)

    def body(j, carry):
        s_start(j)
        s_wait(j - 4)
        return carry

    lax.fori_loop(4, NCH, body, 0)
    for j in range(NCH - 4, NCH):
        s_wait(j)
    plsc.subcore_barrier()
    pltpu.sync_copy(deg_sp.at[pl.ds(sid * RPS, RPS)],
                    out_hbm.at[cid, pl.ds(sid * RPS, RPS)])


CHE = 128
NROW = E // CHE
MAXJ = 79


@functools.partial(
    pl.kernel,
    mesh=_sc_mesh(),
    compiler_params=_SC_PARAMS,
    out_type=jax.ShapeDtypeStruct((2, NP, H), jnp.float32),
    scratch_types=[
        pltpu.VMEM_SHARED((NP, H), jnp.float32),
        pltpu.VMEM_SHARED((NP, H), jnp.float32),
        pltpu.VMEM((MAXJ, CHE), jnp.int32),
        pltpu.VMEM((MAXJ, CHE), jnp.int32),
        pltpu.VMEM((4, CHE, H), jnp.float32),
        pltpu.SemaphoreType.DMA((4,)),
        pltpu.SemaphoreType.DMA((4,)),
        pltpu.SemaphoreType.DMA,
    ],
)
def _edge_scatter_sc(t_hbm, src_hbm, dst_hbm, out_hbm,
                     acc_sp, t_sp, sidx_v, didx_v, rows_v, gsem, ssem, isem):
    cid = lax.axis_index("c")
    sid = lax.axis_index("s")
    wid = cid * 16 + sid
    start = 78 * wid + jnp.maximum(wid - 28, 0)
    nj = 78 + (wid >= 28).astype(jnp.int32)
    l1 = pltpu.async_copy(src_hbm.at[pl.ds(start, MAXJ)], sidx_v, isem)
    l2 = pltpu.async_copy(dst_hbm.at[pl.ds(start, MAXJ)], didx_v, isem)
    l3 = pltpu.async_copy(t_hbm.at[pl.ds(sid * RPS, RPS)],
                          acc_sp.at[pl.ds(sid * RPS, RPS)], isem)
    l4 = pltpu.async_copy(t_hbm.at[pl.ds(sid * RPS, RPS)],
                          t_sp.at[pl.ds(sid * RPS, RPS)], isem)
    l1.wait()
    l2.wait()
    l3.wait()
    l4.wait()
    plsc.subcore_barrier()

    def g_start(j, b):
        pltpu.async_copy(t_sp.at[sidx_v.at[j]], rows_v.at[b], gsem.at[b])

    def g_wait(j, b):
        pltpu.make_async_copy(t_sp.at[sidx_v.at[j]], rows_v.at[b],
                              gsem.at[b]).wait()

    def s_start(j, b):
        pltpu.async_copy(rows_v.at[b], acc_sp.at[didx_v.at[j]], ssem.at[b],
                         add=True)

    def s_wait(j, b):
        pltpu.make_async_copy(rows_v.at[b], acc_sp.at[didx_v.at[j]],
                              ssem.at[b]).wait()

    for j in range(4):
        g_start(j, j)
    g_wait(0, 0)
    s_start(0, 0)

    def body(j, carry):
        b = j % 4
        s_wait(j - 1, (j - 1) % 4)

        @pl.when(j + 3 < nj)
        def _():
            g_start(j + 3, (j + 3) % 4)

        g_wait(j, b)
        s_start(j, b)
        return carry

    lax.fori_loop(1, nj, body, 0)
    s_wait(nj - 1, (nj - 1) % 4)
    plsc.subcore_barrier()
    pltpu.sync_copy(acc_sp.at[pl.ds(sid * RPS, RPS)],
                    out_hbm.at[cid, pl.ds(sid * RPS, RPS)])


def _tc_in_body(x_ref, w_ref, dsum_ref, t_ref):
    dinv = lax.rsqrt(dsum_ref[...])
    h = jnp.dot(x_ref[...], w_ref[...], preferred_element_type=jnp.float32)
    t_ref[pl.ds(0, N), :] = h * dinv[:N]
    t_ref[pl.ds(N, NP - N), :] = jnp.zeros((NP - N, H), jnp.float32)


def _tc_mid_body(s_ref, t1_ref, dsum_ref, b_ref, t2_ref):
    dinv = lax.rsqrt(dsum_ref[...])
    pre = dinv * (s_ref[0] + s_ref[1] - t1_ref[...]) + b_ref[...]
    t2_ref[...] = dinv * jnp.maximum(pre, 0.0)


def _tc_out_body(s_ref, t2_ref, dsum_ref, w_ref, b_ref, o_ref):
    dinv = lax.rsqrt(dsum_ref[...])
    agg = dinv * (s_ref[0] + s_ref[1] - t2_ref[...])
    z = jnp.dot(agg[:N], w_ref[...], preferred_element_type=jnp.float32) + b_ref[...]
    m = jnp.max(z, axis=1, keepdims=True)
    lse = m + jnp.log(jnp.sum(jnp.exp(z - m), axis=1, keepdims=True))
    o_ref[...] = z - lse


_tc_in = pl.pallas_call(
    _tc_in_body, out_shape=jax.ShapeDtypeStruct((NP, H), jnp.float32))
_tc_mid = pl.pallas_call(
    _tc_mid_body, out_shape=jax.ShapeDtypeStruct((NP, H), jnp.float32))
_tc_out = pl.pallas_call(
    _tc_out_body, out_shape=jax.ShapeDtypeStruct((N, C), jnp.float32))


def kernel(x, edge_index, W1, b1, W2, b2):
    ei = edge_index.astype(jnp.int32)
    dst_deg = ei[1].reshape(NW, NCH, CH)
    src = ei[0].reshape(NROW, CHE)
    dst = ei[1].reshape(NROW, CHE)

    degs = _deg_sc(dst_deg)
    dsum = (degs[0] + degs[1] + 1.0).reshape(NP, 1)

    t1 = _tc_in(x, W1, dsum)
    s1 = _edge_scatter_sc(t1, src, dst)
    t2 = _tc_mid(s1, t1, dsum, b1.reshape(1, H))
    s2 = _edge_scatter_sc(t2, src, dst)
    return _tc_out(s2, t2, dsum, W2, b2.reshape(1, C))


# speedup vs baseline: 75.6722x; 1.1452x over previous
import functools

import jax
import jax.numpy as jnp
from jax import lax
from jax.experimental import pallas as pl
from jax.experimental.pallas import tpu as pltpu
from jax.experimental.pallas import tpu_sc as plsc

N = 10000
NP = 10240
E = 320000
D = 128
H = 16
C = 64

RPS = NP // 16
CHE = 128
NROW = E // CHE
MAXJ = 79

_MESH = dict(core_axis_name="c", subcore_axis_name="s")


def _sc_mesh():
    return plsc.VectorSubcoreMesh(**_MESH)


_SC_PARAMS = pltpu.CompilerParams(use_tc_tiling_on_sc=False)


def _tile_range(wid):
    start = 78 * wid + jnp.maximum(wid - 28, 0)
    nj = 78 + (wid >= 28).astype(jnp.int32)
    return start, nj


@functools.partial(
    pl.kernel,
    mesh=_sc_mesh(),
    compiler_params=_SC_PARAMS,
    out_type=jax.ShapeDtypeStruct((2, NP), jnp.float32),
    scratch_types=[
        pltpu.VMEM_SHARED((NP,), jnp.float32),
        pltpu.VMEM((MAXJ, CHE), jnp.int32),
        pltpu.VMEM((CHE,), jnp.float32),
        pltpu.VMEM((RPS,), jnp.float32),
        pltpu.SemaphoreType.DMA,
        pltpu.SemaphoreType.DMA,
    ],
)
def _deg_sc(edge_hbm, out_hbm, deg_sp, didx_v, ones_v, zero_v, isem, ssem):
    cid = lax.axis_index("c")
    sid = lax.axis_index("s")
    wid = cid * 16 + sid
    start, nj = _tile_range(wid)
    ld = pltpu.async_copy(edge_hbm.at[1, pl.ds(start, MAXJ)], didx_v, isem)
    for i in range(RPS // 16):
        zero_v[pl.ds(i * 16, 16)] = jnp.zeros((16,), jnp.float32)
    for i in range(CHE // 16):
        ones_v[pl.ds(i * 16, 16)] = jnp.full((16,), 1.0, jnp.float32)
    pltpu.sync_copy(zero_v, deg_sp.at[pl.ds(sid * RPS, RPS)])
    ld.wait()
    plsc.subcore_barrier()

    def s_start(j):
        pltpu.async_copy(ones_v, deg_sp.at[didx_v.at[j]], ssem, add=True)

    def s_wait(j):
        pltpu.make_async_copy(ones_v, deg_sp.at[didx_v.at[j]], ssem).wait()

    for j in range(4):
        s_start(j)

    def body(j, carry):
        s_start(j)
        s_wait(j - 4)
        return carry

    lax.fori_loop(4, nj, body, 0)
    for j in range(4):
        s_wait(j)
    plsc.subcore_barrier()
    pltpu.sync_copy(deg_sp.at[pl.ds(sid * RPS, RPS)],
                    out_hbm.at[cid, pl.ds(sid * RPS, RPS)])


def _edge_loop(nj, t_sp, acc_sp, sidx_v, didx_v, rows_v, gsem, ssem):
    def g_start(j, b):
        pltpu.async_copy(t_sp.at[sidx_v.at[j]], rows_v.at[b], gsem.at[b])

    def g_wait(j, b):
        pltpu.make_async_copy(t_sp.at[sidx_v.at[j]], rows_v.at[b],
                              gsem.at[b]).wait()

    def s_start(j, b):
        pltpu.async_copy(rows_v.at[b], acc_sp.at[didx_v.at[j]], ssem.at[b],
                         add=True)

    def s_wait(j, b):
        pltpu.make_async_copy(rows_v.at[b], acc_sp.at[didx_v.at[j]],
                              ssem.at[b]).wait()

    for j in range(4):
        g_start(j, j)
    g_wait(0, 0)
    s_start(0, 0)

    def body(j, carry):
        b = j % 4
        s_wait(j - 1, (j - 1) % 4)

        @pl.when(j + 3 < nj)
        def _():
            g_start(j + 3, (j + 3) % 4)

        g_wait(j, b)
        s_start(j, b)
        return carry

    lax.fori_loop(1, nj, body, 0)
    s_wait(nj - 1, (nj - 1) % 4)


@functools.partial(
    pl.kernel,
    mesh=_sc_mesh(),
    compiler_params=_SC_PARAMS,
    out_type=jax.ShapeDtypeStruct((2, NP, H), jnp.float32),
    scratch_types=[
        pltpu.VMEM_SHARED((NP, H), jnp.float32),
        pltpu.VMEM_SHARED((NP, H), jnp.float32),
        pltpu.VMEM((MAXJ, CHE), jnp.int32),
        pltpu.VMEM((MAXJ, CHE), jnp.int32),
        pltpu.VMEM((4, CHE, H), jnp.float32),
        pltpu.SemaphoreType.DMA((4,)),
        pltpu.SemaphoreType.DMA((4,)),
        pltpu.SemaphoreType.DMA,
    ],
)
def _edge_scatter_sc(t_hbm, edge_hbm, out_hbm,
                     acc_sp, t_sp, sidx_v, didx_v, rows_v, gsem, ssem, isem):
    cid = lax.axis_index("c")
    sid = lax.axis_index("s")
    wid = cid * 16 + sid
    start, nj = _tile_range(wid)
    l1 = pltpu.async_copy(edge_hbm.at[0, pl.ds(start, MAXJ)], sidx_v, isem)
    l2 = pltpu.async_copy(edge_hbm.at[1, pl.ds(start, MAXJ)], didx_v, isem)
    l3 = pltpu.async_copy(t_hbm.at[pl.ds(sid * RPS, RPS)],
                          acc_sp.at[pl.ds(sid * RPS, RPS)], isem)
    l4 = pltpu.async_copy(t_hbm.at[pl.ds(sid * RPS, RPS)],
                          t_sp.at[pl.ds(sid * RPS, RPS)], isem)
    l1.wait()
    l2.wait()
    l3.wait()
    l4.wait()
    plsc.subcore_barrier()
    _edge_loop(nj, t_sp, acc_sp, sidx_v, didx_v, rows_v, gsem, ssem)
    plsc.subcore_barrier()
    pltpu.sync_copy(acc_sp.at[pl.ds(sid * RPS, RPS)],
                    out_hbm.at[cid, pl.ds(sid * RPS, RPS)])


@functools.partial(
    pl.kernel,
    mesh=_sc_mesh(),
    compiler_params=_SC_PARAMS,
    out_type=(jax.ShapeDtypeStruct((2, NP, H), jnp.float32),
              jax.ShapeDtypeStruct((NP, H), jnp.float32)),
    scratch_types=[
        pltpu.VMEM_SHARED((NP, H), jnp.float32),
        pltpu.VMEM_SHARED((NP, H), jnp.float32),
        pltpu.VMEM((MAXJ, CHE), jnp.int32),
        pltpu.VMEM((MAXJ, CHE), jnp.int32),
        pltpu.VMEM((4, CHE, H), jnp.float32),
        pltpu.VMEM((RPS, H), jnp.float32),
        pltpu.VMEM((RPS, H), jnp.float32),
        pltpu.VMEM((RPS, H), jnp.float32),
        pltpu.VMEM((RPS, H), jnp.float32),
        pltpu.VMEM((RPS, H), jnp.float32),
        pltpu.VMEM((H,), jnp.float32),
        pltpu.SemaphoreType.DMA((4,)),
        pltpu.SemaphoreType.DMA((4,)),
        pltpu.SemaphoreType.DMA,
    ],
)
def _edge_scatter_mid_sc(s1_hbm, t1_hbm, dv_hbm, b1_hbm, edge_hbm,
                         out_hbm, t2_hbm,
                         acc_sp, t_sp, sidx_v, didx_v, rows_v,
                         s1a_v, s1b_v, t1_v, dv_v, t2_v, b1_v,
                         gsem, ssem, isem):
    cid = lax.axis_index("c")
    sid = lax.axis_index("s")
    wid = cid * 16 + sid
    start, nj = _tile_range(wid)
    sl = pl.ds(sid * RPS, RPS)
    loads = [
        pltpu.async_copy(edge_hbm.at[0, pl.ds(start, MAXJ)], sidx_v, isem),
        pltpu.async_copy(edge_hbm.at[1, pl.ds(start, MAXJ)], didx_v, isem),
        pltpu.async_copy(s1_hbm.at[0, sl], s1a_v, isem),
        pltpu.async_copy(s1_hbm.at[1, sl], s1b_v, isem),
        pltpu.async_copy(t1_hbm.at[sl], t1_v, isem),
        pltpu.async_copy(dv_hbm.at[sl], dv_v, isem),
        pltpu.async_copy(b1_hbm, b1_v, isem),
    ]
    for l in loads:
        l.wait()

    b1r = b1_v[...]

    def mid(r, carry):
        d = dv_v[r]
        pre = d * (s1a_v[r] + s1b_v[r] - t1_v[r]) + b1r
        t2_v[r] = d * jnp.maximum(pre, 0.0)
        return carry

    lax.fori_loop(0, RPS, mid, 0, unroll=8)

    m1 = pltpu.async_copy(t2_v, acc_sp.at[sl], isem)
    m2 = pltpu.async_copy(t2_v, t_sp.at[sl], isem)
    m1.wait()
    m2.wait()

    @pl.when(cid == 0)
    def _():
        pltpu.sync_copy(t2_v, t2_hbm.at[sl])

    plsc.subcore_barrier()
    _edge_loop(nj, t_sp, acc_sp, sidx_v, didx_v, rows_v, gsem, ssem)
    plsc.subcore_barrier()
    pltpu.sync_copy(acc_sp.at[sl], out_hbm.at[cid, sl])


def _tc_in_body(x_ref, w_ref, dsum_ref, t_ref, dv_ref):
    dinv = lax.rsqrt(dsum_ref[...])
    h = jnp.dot(x_ref[...], w_ref[...], preferred_element_type=jnp.float32)
    t_ref[pl.ds(0, N), :] = h * dinv[:N]
    t_ref[pl.ds(N, NP - N), :] = jnp.zeros((NP - N, H), jnp.float32)
    dv_ref[...] = jnp.broadcast_to(dinv, (NP, H))


def _tc_out_body(s_ref, t2_ref, dsum_ref, w_ref, b_ref, o_ref):
    dinv = lax.rsqrt(dsum_ref[...])
    agg = dinv * (s_ref[0] + s_ref[1] - t2_ref[...])
    z = jnp.dot(agg[:N], w_ref[...], preferred_element_type=jnp.float32) + b_ref[...]
    m = jnp.max(z, axis=1, keepdims=True)
    lse = m + jnp.log(jnp.sum(jnp.exp(z - m), axis=1, keepdims=True))
    o_ref[...] = z - lse


_tc_in = pl.pallas_call(
    _tc_in_body,
    out_shape=(jax.ShapeDtypeStruct((NP, H), jnp.float32),
               jax.ShapeDtypeStruct((NP, H), jnp.float32)))
_tc_out = pl.pallas_call(
    _tc_out_body, out_shape=jax.ShapeDtypeStruct((N, C), jnp.float32))


def kernel(x, edge_index, W1, b1, W2, b2):
    edges = edge_index.astype(jnp.int32).reshape(2, NROW, CHE)

    degs = _deg_sc(edges)
    dsum = (degs[0] + degs[1] + 1.0).reshape(NP, 1)

    t1, dv16 = _tc_in(x, W1, dsum)
    s1 = _edge_scatter_sc(t1, edges)
    s2, t2 = _edge_scatter_mid_sc(s1, t1, dv16, b1, edges)
    return _tc_out(s2, t2, dsum, W2, b2.reshape(1, C))


# speedup vs baseline: 78.9232x; 1.0430x over previous
import functools

import jax
import jax.numpy as jnp
from jax import lax
from jax.experimental import pallas as pl
from jax.experimental.pallas import tpu as pltpu
from jax.experimental.pallas import tpu_sc as plsc

N = 10000
NP = 10240
E = 320000
D = 128
H = 16
C = 64

RPS = NP // 16
CHE = 128
NROW = E // CHE
MAXJ = 79

_MESH = dict(core_axis_name="c", subcore_axis_name="s")


def _sc_mesh():
    return plsc.VectorSubcoreMesh(**_MESH)


_SC_PARAMS = pltpu.CompilerParams(use_tc_tiling_on_sc=False)


def _tile_range(wid):
    start = 78 * wid + jnp.maximum(wid - 28, 0)
    nj = 78 + (wid >= 28).astype(jnp.int32)
    return start, nj


@functools.partial(
    pl.kernel,
    mesh=_sc_mesh(),
    compiler_params=_SC_PARAMS,
    out_type=jax.ShapeDtypeStruct((2, NP), jnp.float32),
    scratch_types=[
        pltpu.VMEM_SHARED((NP,), jnp.float32),
        pltpu.VMEM((MAXJ, CHE), jnp.int32),
        pltpu.VMEM((CHE,), jnp.float32),
        pltpu.VMEM((RPS,), jnp.float32),
        pltpu.SemaphoreType.DMA,
        pltpu.SemaphoreType.DMA,
    ],
)
def _deg_sc(edge_hbm, out_hbm, deg_sp, didx_v, ones_v, zero_v, isem, ssem):
    cid = lax.axis_index("c")
    sid = lax.axis_index("s")
    wid = cid * 16 + sid
    start, nj = _tile_range(wid)
    ld = pltpu.async_copy(edge_hbm.at[1, pl.ds(start, MAXJ)], didx_v, isem)
    for i in range(RPS // 16):
        zero_v[pl.ds(i * 16, 16)] = jnp.zeros((16,), jnp.float32)
    for i in range(CHE // 16):
        ones_v[pl.ds(i * 16, 16)] = jnp.full((16,), 1.0, jnp.float32)
    pltpu.sync_copy(zero_v, deg_sp.at[pl.ds(sid * RPS, RPS)])
    ld.wait()
    plsc.subcore_barrier()

    def s_start(j):
        pltpu.async_copy(ones_v, deg_sp.at[didx_v.at[j]], ssem, add=True)

    def s_wait(j):
        pltpu.make_async_copy(ones_v, deg_sp.at[didx_v.at[j]], ssem).wait()

    for j in range(4):
        s_start(j)

    def body(j, carry):
        s_start(j)
        s_wait(j - 4)
        return carry

    lax.fori_loop(4, nj, body, 0)
    for j in range(4):
        s_wait(j)
    plsc.subcore_barrier()
    pltpu.sync_copy(deg_sp.at[pl.ds(sid * RPS, RPS)],
                    out_hbm.at[cid, pl.ds(sid * RPS, RPS)])


def _edge_loop(nj, t_sp, acc_sp, sidx_v, didx_v, rows_v, gsem, ssem):
    def g_start(j, b):
        pltpu.async_copy(t_sp.at[sidx_v.at[j]], rows_v.at[b], gsem.at[b])

    def g_wait(j, b):
        pltpu.make_async_copy(t_sp.at[sidx_v.at[j]], rows_v.at[b],
                              gsem.at[b]).wait()

    def s_start(j, b):
        pltpu.async_copy(rows_v.at[b], acc_sp.at[didx_v.at[j]], ssem.at[b],
                         add=True)

    def s_wait(j, b):
        pltpu.make_async_copy(rows_v.at[b], acc_sp.at[didx_v.at[j]],
                              ssem.at[b]).wait()

    for j in range(4):
        g_start(j, j)
    g_wait(0, 0)
    s_start(0, 0)

    def body(j, carry):
        b = j % 4
        s_wait(j - 1, (j - 1) % 4)

        @pl.when(j + 3 < nj)
        def _():
            g_start(j + 3, (j + 3) % 4)

        g_wait(j, b)
        s_start(j, b)
        return carry

    lax.fori_loop(1, nj, body, 0)
    s_wait(nj - 1, (nj - 1) % 4)


@functools.partial(
    pl.kernel,
    mesh=_sc_mesh(),
    compiler_params=_SC_PARAMS,
    out_type=jax.ShapeDtypeStruct((2, NP, H), jnp.float32),
    scratch_types=[
        pltpu.VMEM_SHARED((NP, H), jnp.float32),
        pltpu.VMEM_SHARED((NP, H), jnp.float32),
        pltpu.VMEM((MAXJ, CHE), jnp.int32),
        pltpu.VMEM((MAXJ, CHE), jnp.int32),
        pltpu.VMEM((4, CHE, H), jnp.float32),
        pltpu.SemaphoreType.DMA((4,)),
        pltpu.SemaphoreType.DMA((4,)),
        pltpu.SemaphoreType.DMA,
    ],
)
def _edge_scatter_sc(t_hbm, edge_hbm, out_hbm,
                     acc_sp, t_sp, sidx_v, didx_v, rows_v, gsem, ssem, isem):
    cid = lax.axis_index("c")
    sid = lax.axis_index("s")
    wid = cid * 16 + sid
    start, nj = _tile_range(wid)
    l1 = pltpu.async_copy(edge_hbm.at[0, pl.ds(start, MAXJ)], sidx_v, isem)
    l2 = pltpu.async_copy(edge_hbm.at[1, pl.ds(start, MAXJ)], didx_v, isem)
    l4 = pltpu.async_copy(t_hbm.at[pl.ds(sid * RPS, RPS)],
                          t_sp.at[pl.ds(sid * RPS, RPS)], isem)

    @pl.when(cid == 0)
    def _():
        pltpu.sync_copy(t_hbm.at[pl.ds(sid * RPS, RPS)],
                        acc_sp.at[pl.ds(sid * RPS, RPS)])

    @pl.when(cid == 1)
    def _():
        for i in range(CHE):
            rows_v[0, i] = jnp.zeros((H,), jnp.float32)
        for k in range(RPS // CHE):
            pltpu.sync_copy(rows_v.at[0],
                            acc_sp.at[pl.ds(sid * RPS + k * CHE, CHE)])

    l1.wait()
    l2.wait()
    l4.wait()
    plsc.subcore_barrier()
    _edge_loop(nj, t_sp, acc_sp, sidx_v, didx_v, rows_v, gsem, ssem)
    plsc.subcore_barrier()
    pltpu.sync_copy(acc_sp.at[pl.ds(sid * RPS, RPS)],
                    out_hbm.at[cid, pl.ds(sid * RPS, RPS)])


@functools.partial(
    pl.kernel,
    mesh=_sc_mesh(),
    compiler_params=_SC_PARAMS,
    out_type=jax.ShapeDtypeStruct((2, NP, H), jnp.float32),
    scratch_types=[
        pltpu.VMEM_SHARED((NP, H), jnp.float32),
        pltpu.VMEM_SHARED((NP, H), jnp.float32),
        pltpu.VMEM((MAXJ, CHE), jnp.int32),
        pltpu.VMEM((MAXJ, CHE), jnp.int32),
        pltpu.VMEM((4, CHE, H), jnp.float32),
        pltpu.VMEM((RPS, H), jnp.float32),
        pltpu.VMEM((RPS, H), jnp.float32),
        pltpu.VMEM((RPS, H), jnp.float32),
        pltpu.VMEM((RPS, H), jnp.float32),
        pltpu.VMEM((H,), jnp.float32),
        pltpu.SemaphoreType.DMA((4,)),
        pltpu.SemaphoreType.DMA((4,)),
        pltpu.SemaphoreType.DMA,
    ],
)
def _edge_scatter_mid_sc(s1_hbm, dv_hbm, b1_hbm, edge_hbm, out_hbm,
                         acc_sp, t_sp, sidx_v, didx_v, rows_v,
                         s1a_v, s1b_v, dv_v, t2_v, b1_v,
                         gsem, ssem, isem):
    cid = lax.axis_index("c")
    sid = lax.axis_index("s")
    wid = cid * 16 + sid
    start, nj = _tile_range(wid)
    sl = pl.ds(sid * RPS, RPS)
    loads = [
        pltpu.async_copy(edge_hbm.at[0, pl.ds(start, MAXJ)], sidx_v, isem),
        pltpu.async_copy(edge_hbm.at[1, pl.ds(start, MAXJ)], didx_v, isem),
        pltpu.async_copy(s1_hbm.at[0, sl], s1a_v, isem),
        pltpu.async_copy(s1_hbm.at[1, sl], s1b_v, isem),
        pltpu.async_copy(dv_hbm.at[sl], dv_v, isem),
        pltpu.async_copy(b1_hbm, b1_v, isem),
    ]
    for l in loads:
        l.wait()

    b1r = b1_v[...]

    def mid(r, carry):
        d = dv_v[r]
        pre = d * (s1a_v[r] + s1b_v[r]) + b1r
        t2_v[r] = d * jnp.maximum(pre, 0.0)
        return carry

    lax.fori_loop(0, RPS, mid, 0, unroll=16)

    m2 = pltpu.async_copy(t2_v, t_sp.at[sl], isem)

    @pl.when(cid == 0)
    def _():
        pltpu.sync_copy(t2_v, acc_sp.at[sl])

    @pl.when(cid == 1)
    def _():
        for i in range(CHE):
            rows_v[0, i] = jnp.zeros((H,), jnp.float32)
        for k in range(RPS // CHE):
            pltpu.sync_copy(rows_v.at[0],
                            acc_sp.at[pl.ds(sid * RPS + k * CHE, CHE)])

    m2.wait()
    plsc.subcore_barrier()
    _edge_loop(nj, t_sp, acc_sp, sidx_v, didx_v, rows_v, gsem, ssem)
    plsc.subcore_barrier()
    pltpu.sync_copy(acc_sp.at[sl], out_hbm.at[cid, sl])


def _tc_in_body(x_ref, w_ref, dsum_ref, t_ref, dv_ref):
    dinv = lax.rsqrt(dsum_ref[...])
    h = jnp.dot(x_ref[...], w_ref[...], preferred_element_type=jnp.float32)
    t_ref[pl.ds(0, N), :] = h * dinv[:N]
    t_ref[pl.ds(N, NP - N), :] = jnp.zeros((NP - N, H), jnp.float32)
    dv_ref[...] = jnp.broadcast_to(dinv, (NP, H))


def _tc_out_body(s_ref, dsum_ref, w_ref, b_ref, o_ref):
    agg = lax.rsqrt(dsum_ref[...]) * (s_ref[0] + s_ref[1])
    z = jnp.dot(agg[:N], w_ref[...], preferred_element_type=jnp.float32) + b_ref[...]
    m = jnp.max(z, axis=1, keepdims=True)
    lse = m + jnp.log(jnp.sum(jnp.exp(z - m), axis=1, keepdims=True))
    o_ref[...] = z - lse


_tc_in = pl.pallas_call(
    _tc_in_body,
    out_shape=(jax.ShapeDtypeStruct((NP, H), jnp.float32),
               jax.ShapeDtypeStruct((NP, H), jnp.float32)))
_tc_out = pl.pallas_call(
    _tc_out_body, out_shape=jax.ShapeDtypeStruct((N, C), jnp.float32))


def kernel(x, edge_index, W1, b1, W2, b2):
    edges = edge_index.astype(jnp.int32).reshape(2, NROW, CHE)

    degs = _deg_sc(edges)
    dsum = (degs[0] + degs[1] + 1.0).reshape(NP, 1)

    t1, dv16 = _tc_in(x, W1, dsum)
    s1 = _edge_scatter_sc(t1, edges)
    s2 = _edge_scatter_mid_sc(s1, dv16, b1, edges)
    return _tc_out(s2, dsum, W2, b2.reshape(1, C))


# speedup vs baseline: 82.5415x; 1.0458x over previous
import functools

import jax
import jax.numpy as jnp
from jax import lax
from jax.experimental import pallas as pl
from jax.experimental.pallas import tpu as pltpu
from jax.experimental.pallas import tpu_sc as plsc

N = 10000
NP = 10240
E = 320000
D = 128
H = 16
C = 64

RPS = NP // 16
CHE = 128
NROW = E // CHE
MAXJ = 79

_MESH = dict(core_axis_name="c", subcore_axis_name="s")


def _sc_mesh():
    return plsc.VectorSubcoreMesh(**_MESH)


_SC_PARAMS = pltpu.CompilerParams(use_tc_tiling_on_sc=False)


def _tile_range(wid):
    start = 78 * wid + jnp.maximum(wid - 28, 0)
    nj = 78 + (wid >= 28).astype(jnp.int32)
    return start, nj


@functools.partial(
    pl.kernel,
    mesh=_sc_mesh(),
    compiler_params=_SC_PARAMS,
    out_type=jax.ShapeDtypeStruct((2, NP), jnp.float32),
    scratch_types=[
        pltpu.VMEM_SHARED((NP,), jnp.float32),
        pltpu.VMEM((MAXJ, CHE), jnp.int32),
        pltpu.VMEM((CHE,), jnp.float32),
        pltpu.VMEM((RPS,), jnp.float32),
        pltpu.SemaphoreType.DMA,
        pltpu.SemaphoreType.DMA,
    ],
)
def _deg_sc(edge_hbm, out_hbm, deg_sp, didx_v, ones_v, zero_v, isem, ssem):
    cid = lax.axis_index("c")
    sid = lax.axis_index("s")
    wid = cid * 16 + sid
    start, nj = _tile_range(wid)
    ld = pltpu.async_copy(edge_hbm.at[1, pl.ds(start, MAXJ)], didx_v, isem)
    for i in range(RPS // 16):
        zero_v[pl.ds(i * 16, 16)] = jnp.zeros((16,), jnp.float32)
    for i in range(CHE // 16):
        ones_v[pl.ds(i * 16, 16)] = jnp.full((16,), 1.0, jnp.float32)
    pltpu.sync_copy(zero_v, deg_sp.at[pl.ds(sid * RPS, RPS)])
    ld.wait()
    plsc.subcore_barrier()

    def s_start(j):
        pltpu.async_copy(ones_v, deg_sp.at[didx_v.at[j]], ssem, add=True)

    def s_wait(j):
        pltpu.make_async_copy(ones_v, deg_sp.at[didx_v.at[j]], ssem).wait()

    for j in range(4):
        s_start(j)

    def body(j, carry):
        s_start(j)
        s_wait(j - 4)
        return carry

    lax.fori_loop(4, nj, body, 0)
    for j in range(4):
        s_wait(j)
    plsc.subcore_barrier()
    pltpu.sync_copy(deg_sp.at[pl.ds(sid * RPS, RPS)],
                    out_hbm.at[cid, pl.ds(sid * RPS, RPS)])


RING = 8


def _edge_loop(nj, t_sp, acc_sp, sidx_v, didx_v, rows_v, gsem, ssem):
    def g_start(j, b):
        pltpu.async_copy(t_sp.at[sidx_v.at[j]], rows_v.at[b], gsem.at[b])

    def g_wait(j, b):
        pltpu.make_async_copy(t_sp.at[sidx_v.at[j]], rows_v.at[b],
                              gsem.at[b]).wait()

    def s_start(j, b):
        pltpu.async_copy(rows_v.at[b], acc_sp.at[didx_v.at[j]], ssem.at[b],
                         add=True)

    def s_wait(j, b):
        pltpu.make_async_copy(rows_v.at[b], acc_sp.at[didx_v.at[j]],
                              ssem.at[b]).wait()

    for j in range(RING):
        g_start(j, j)
    g_wait(0, 0)
    s_start(0, 0)

    def body(j, carry):
        b = j % RING
        s_wait(j - 1, (j - 1) % RING)

        @pl.when(j + RING - 1 < nj)
        def _():
            g_start(j + RING - 1, (j + RING - 1) % RING)

        g_wait(j, b)
        s_start(j, b)
        return carry

    lax.fori_loop(1, nj, body, 0)
    s_wait(nj - 1, (nj - 1) % RING)


@functools.partial(
    pl.kernel,
    mesh=_sc_mesh(),
    compiler_params=_SC_PARAMS,
    out_type=jax.ShapeDtypeStruct((2, NP, H), jnp.float32),
    scratch_types=[
        pltpu.VMEM_SHARED((NP, H), jnp.float32),
        pltpu.VMEM_SHARED((NP, H), jnp.float32),
        pltpu.VMEM((MAXJ, CHE), jnp.int32),
        pltpu.VMEM((MAXJ, CHE), jnp.int32),
        pltpu.VMEM((RING, CHE, H), jnp.float32),
        pltpu.SemaphoreType.DMA((RING,)),
        pltpu.SemaphoreType.DMA((RING,)),
        pltpu.SemaphoreType.DMA,
    ],
)
def _edge_scatter_sc(t_hbm, edge_hbm, out_hbm,
                     acc_sp, t_sp, sidx_v, didx_v, rows_v, gsem, ssem, isem):
    cid = lax.axis_index("c")
    sid = lax.axis_index("s")
    wid = cid * 16 + sid
    start, nj = _tile_range(wid)
    l1 = pltpu.async_copy(edge_hbm.at[0, pl.ds(start, MAXJ)], sidx_v, isem)
    l2 = pltpu.async_copy(edge_hbm.at[1, pl.ds(start, MAXJ)], didx_v, isem)
    l4 = pltpu.async_copy(t_hbm.at[pl.ds(sid * RPS, RPS)],
                          t_sp.at[pl.ds(sid * RPS, RPS)], isem)

    @pl.when(cid == 0)
    def _():
        pltpu.sync_copy(t_hbm.at[pl.ds(sid * RPS, RPS)],
                        acc_sp.at[pl.ds(sid * RPS, RPS)])

    @pl.when(cid == 1)
    def _():
        for i in range(CHE):
            rows_v[0, i] = jnp.zeros((H,), jnp.float32)
        for k in range(RPS // CHE):
            pltpu.sync_copy(rows_v.at[0],
                            acc_sp.at[pl.ds(sid * RPS + k * CHE, CHE)])

    l1.wait()
    l2.wait()
    l4.wait()
    plsc.subcore_barrier()
    _edge_loop(nj, t_sp, acc_sp, sidx_v, didx_v, rows_v, gsem, ssem)
    plsc.subcore_barrier()
    pltpu.sync_copy(acc_sp.at[pl.ds(sid * RPS, RPS)],
                    out_hbm.at[cid, pl.ds(sid * RPS, RPS)])


@functools.partial(
    pl.kernel,
    mesh=_sc_mesh(),
    compiler_params=_SC_PARAMS,
    out_type=jax.ShapeDtypeStruct((2, NP, H), jnp.float32),
    scratch_types=[
        pltpu.VMEM_SHARED((NP, H), jnp.float32),
        pltpu.VMEM_SHARED((NP, H), jnp.float32),
        pltpu.VMEM((MAXJ, CHE), jnp.int32),
        pltpu.VMEM((MAXJ, CHE), jnp.int32),
        pltpu.VMEM((RING, CHE, H), jnp.float32),
        pltpu.VMEM((RPS, H), jnp.float32),
        pltpu.VMEM((RPS, H), jnp.float32),
        pltpu.VMEM((RPS, H), jnp.float32),
        pltpu.VMEM((RPS, H), jnp.float32),
        pltpu.VMEM((H,), jnp.float32),
        pltpu.SemaphoreType.DMA((RING,)),
        pltpu.SemaphoreType.DMA((RING,)),
        pltpu.SemaphoreType.DMA,
    ],
)
def _edge_scatter_mid_sc(s1_hbm, dv_hbm, b1_hbm, edge_hbm, out_hbm,
                         acc_sp, t_sp, sidx_v, didx_v, rows_v,
                         s1a_v, s1b_v, dv_v, t2_v, b1_v,
                         gsem, ssem, isem):
    cid = lax.axis_index("c")
    sid = lax.axis_index("s")
    wid = cid * 16 + sid
    start, nj = _tile_range(wid)
    sl = pl.ds(sid * RPS, RPS)
    loads = [
        pltpu.async_copy(edge_hbm.at[0, pl.ds(start, MAXJ)], sidx_v, isem),
        pltpu.async_copy(edge_hbm.at[1, pl.ds(start, MAXJ)], didx_v, isem),
        pltpu.async_copy(s1_hbm.at[0, sl], s1a_v, isem),
        pltpu.async_copy(s1_hbm.at[1, sl], s1b_v, isem),
        pltpu.async_copy(dv_hbm.at[sl], dv_v, isem),
        pltpu.async_copy(b1_hbm, b1_v, isem),
    ]
    for l in loads:
        l.wait()

    b1r = b1_v[...]

    @plsc.parallel_loop(0, RPS, unroll=8)
    def _mid(r):
        d = dv_v[r]
        pre = d * (s1a_v[r] + s1b_v[r]) + b1r
        t2_v[r] = d * jnp.maximum(pre, 0.0)

    m2 = pltpu.async_copy(t2_v, t_sp.at[sl], isem)

    @pl.when(cid == 0)
    def _():
        pltpu.sync_copy(t2_v, acc_sp.at[sl])

    @pl.when(cid == 1)
    def _():
        for i in range(CHE):
            rows_v[0, i] = jnp.zeros((H,), jnp.float32)
        for k in range(RPS // CHE):
            pltpu.sync_copy(rows_v.at[0],
                            acc_sp.at[pl.ds(sid * RPS + k * CHE, CHE)])

    m2.wait()
    plsc.subcore_barrier()
    _edge_loop(nj, t_sp, acc_sp, sidx_v, didx_v, rows_v, gsem, ssem)
    plsc.subcore_barrier()
    pltpu.sync_copy(acc_sp.at[sl], out_hbm.at[cid, sl])


def _tc_in_body(x_ref, w_ref, dsum_ref, t_ref, dv_ref):
    dinv = lax.rsqrt(dsum_ref[...])
    h = jnp.dot(x_ref[...], w_ref[...], preferred_element_type=jnp.float32)
    t_ref[pl.ds(0, N), :] = h * dinv[:N]
    t_ref[pl.ds(N, NP - N), :] = jnp.zeros((NP - N, H), jnp.float32)
    dv_ref[...] = jnp.broadcast_to(dinv, (NP, H))


def _tc_out_body(s_ref, dsum_ref, w_ref, b_ref, o_ref):
    agg = lax.rsqrt(dsum_ref[...]) * (s_ref[0] + s_ref[1])
    z = jnp.dot(agg[:N], w_ref[...], preferred_element_type=jnp.float32) + b_ref[...]
    m = jnp.max(z, axis=1, keepdims=True)
    lse = m + jnp.log(jnp.sum(jnp.exp(z - m), axis=1, keepdims=True))
    o_ref[...] = z - lse


_tc_in = pl.pallas_call(
    _tc_in_body,
    out_shape=(jax.ShapeDtypeStruct((NP, H), jnp.float32),
               jax.ShapeDtypeStruct((NP, H), jnp.float32)))
_tc_out = pl.pallas_call(
    _tc_out_body, out_shape=jax.ShapeDtypeStruct((N, C), jnp.float32))


def kernel(x, edge_index, W1, b1, W2, b2):
    edges = edge_index.astype(jnp.int32).reshape(2, NROW, CHE)

    degs = _deg_sc(edges)
    dsum = (degs[0] + degs[1] + 1.0).reshape(NP, 1)

    t1, dv16 = _tc_in(x, W1, dsum)
    s1 = _edge_scatter_sc(t1, edges)
    s2 = _edge_scatter_mid_sc(s1, dv16, b1, edges)
    return _tc_out(s2, dsum, W2, b2.reshape(1, C))


# speedup vs baseline: 90.3065x; 1.0941x over previous
import functools

import jax
import jax.numpy as jnp
from jax import lax
from jax.experimental import pallas as pl
from jax.experimental.pallas import tpu as pltpu
from jax.experimental.pallas import tpu_sc as plsc

N = 10000
NP = 10240
E = 320000
D = 128
H = 16
C = 64

RPS = NP // 16
CHE = 128
NROW = E // CHE
MAXJ = 79

_MESH = dict(core_axis_name="c", subcore_axis_name="s")


def _sc_mesh():
    return plsc.VectorSubcoreMesh(**_MESH)


_SC_PARAMS = pltpu.CompilerParams(use_tc_tiling_on_sc=False,
                                  needs_layout_passes=False)


def _tile_range(wid):
    start = 78 * wid + jnp.maximum(wid - 28, 0)
    nj = 78 + (wid >= 28).astype(jnp.int32)
    return start, nj


@functools.partial(
    pl.kernel,
    mesh=_sc_mesh(),
    compiler_params=_SC_PARAMS,
    out_type=jax.ShapeDtypeStruct((2, NP), jnp.float32),
    scratch_types=[
        pltpu.VMEM_SHARED((NP,), jnp.float32),
        pltpu.VMEM((MAXJ, CHE), jnp.int32),
        pltpu.VMEM((CHE,), jnp.float32),
        pltpu.VMEM((RPS,), jnp.float32),
        pltpu.SemaphoreType.DMA,
        pltpu.SemaphoreType.DMA,
    ],
)
def _deg_sc(edge_hbm, out_hbm, deg_sp, didx_v, ones_v, zero_v, isem, ssem):
    cid = lax.axis_index("c")
    sid = lax.axis_index("s")
    wid = cid * 16 + sid
    start, nj = _tile_range(wid)
    ld = pltpu.async_copy(edge_hbm.at[1, pl.ds(start, MAXJ)], didx_v, isem)
    for i in range(RPS // 16):
        zero_v[pl.ds(i * 16, 16)] = jnp.zeros((16,), jnp.float32)
    for i in range(CHE // 16):
        ones_v[pl.ds(i * 16, 16)] = jnp.full((16,), 1.0, jnp.float32)
    pltpu.sync_copy(zero_v, deg_sp.at[pl.ds(sid * RPS, RPS)])
    ld.wait()
    plsc.subcore_barrier()

    def s_start(j):
        pltpu.async_copy(ones_v, deg_sp.at[didx_v.at[j]], ssem, add=True)

    def s_wait(j):
        pltpu.make_async_copy(ones_v, deg_sp.at[didx_v.at[j]], ssem).wait()

    for j in range(4):
        s_start(j)

    def body(j, carry):
        s_start(j)
        s_wait(j - 4)
        return carry

    lax.fori_loop(4, nj, body, 0)
    for j in range(4):
        s_wait(j)
    plsc.subcore_barrier()
    pltpu.sync_copy(deg_sp.at[pl.ds(sid * RPS, RPS)],
                    out_hbm.at[cid, pl.ds(sid * RPS, RPS)])


RING = 8


def _rsqrt_nr(x):
    i = lax.bitcast_convert_type(x, jnp.int32)
    i = jnp.int32(0x5F3759DF) - lax.shift_right_arithmetic(i, 1)
    y = lax.bitcast_convert_type(i, jnp.float32)
    xh = x * 0.5
    for _ in range(3):
        y = y * (1.5 - xh * y * y)
    return y


def _compute_dinv(da_v, db_v, dinv_v):
    for i in range(RPS // 16):
        s = pl.ds(i * 16, 16)
        dinv_v[s] = _rsqrt_nr(da_v[s] + db_v[s] + 1.0)


def _splat(vec_ref, r):
    idx = jnp.zeros((16,), jnp.int32) + r
    return plsc.load_gather(vec_ref, [idx])


def _edge_loop(nj, t_sp, acc_sp, sidx_v, didx_v, rows_v, gsem, ssem):
    def g_start(j, b):
        pltpu.async_copy(t_sp.at[sidx_v.at[j]], rows_v.at[b], gsem.at[b])

    def g_wait(j, b):
        pltpu.make_async_copy(t_sp.at[sidx_v.at[j]], rows_v.at[b],
                              gsem.at[b]).wait()

    def s_start(j, b):
        pltpu.async_copy(rows_v.at[b], acc_sp.at[didx_v.at[j]], ssem.at[b],
                         add=True)

    def s_wait(j, b):
        pltpu.make_async_copy(rows_v.at[b], acc_sp.at[didx_v.at[j]],
                              ssem.at[b]).wait()

    for j in range(RING):
        g_start(j, j)
    g_wait(0, 0)
    s_start(0, 0)

    def body(j, carry):
        b = j % RING
        s_wait(j - 1, (j - 1) % RING)

        @pl.when(j + RING - 1 < nj)
        def _():
            g_start(j + RING - 1, (j + RING - 1) % RING)

        g_wait(j, b)
        s_start(j, b)
        return carry

    lax.fori_loop(1, nj, body, 0)
    s_wait(nj - 1, (nj - 1) % RING)


@functools.partial(
    pl.kernel,
    mesh=_sc_mesh(),
    compiler_params=_SC_PARAMS,
    out_type=jax.ShapeDtypeStruct((2, NP, H), jnp.float32),
    scratch_types=[
        pltpu.VMEM_SHARED((NP, H), jnp.float32),
        pltpu.VMEM_SHARED((NP, H), jnp.float32),
        pltpu.VMEM((MAXJ, CHE), jnp.int32),
        pltpu.VMEM((MAXJ, CHE), jnp.int32),
        pltpu.VMEM((RING, CHE, H), jnp.float32),
        pltpu.VMEM((RPS, H), jnp.float32),
        pltpu.VMEM((RPS, H), jnp.float32),
        pltpu.VMEM((RPS,), jnp.float32),
        pltpu.VMEM((RPS,), jnp.float32),
        pltpu.VMEM((RPS,), jnp.float32),
        pltpu.SemaphoreType.DMA((RING,)),
        pltpu.SemaphoreType.DMA((RING,)),
        pltpu.SemaphoreType.DMA,
    ],
)
def _edge_scale_scatter_sc(h_hbm, degs_hbm, edge_hbm, out_hbm,
                           acc_sp, t_sp, sidx_v, didx_v, rows_v,
                           h_v, t1_v, da_v, db_v, dinv_v, gsem, ssem, isem):
    cid = lax.axis_index("c")
    sid = lax.axis_index("s")
    wid = cid * 16 + sid
    start, nj = _tile_range(wid)
    sl = pl.ds(sid * RPS, RPS)
    loads = [
        pltpu.async_copy(edge_hbm.at[0, pl.ds(start, MAXJ)], sidx_v, isem),
        pltpu.async_copy(edge_hbm.at[1, pl.ds(start, MAXJ)], didx_v, isem),
        pltpu.async_copy(h_hbm.at[sl], h_v, isem),
        pltpu.async_copy(degs_hbm.at[0, sl], da_v, isem),
        pltpu.async_copy(degs_hbm.at[1, sl], db_v, isem),
    ]
    for l in loads:
        l.wait()

    _compute_dinv(da_v, db_v, dinv_v)

    @plsc.parallel_loop(0, RPS, unroll=8)
    def _scale(r):
        t1_v[r] = _splat(dinv_v, r) * h_v[r]

    m2 = pltpu.async_copy(t1_v, t_sp.at[sl], isem)

    @pl.when(cid == 0)
    def _():
        pltpu.sync_copy(t1_v, acc_sp.at[sl])

    @pl.when(cid == 1)
    def _():
        for i in range(CHE):
            rows_v[0, i] = jnp.zeros((H,), jnp.float32)
        for k in range(RPS // CHE):
            pltpu.sync_copy(rows_v.at[0],
                            acc_sp.at[pl.ds(sid * RPS + k * CHE, CHE)])

    m2.wait()
    plsc.subcore_barrier()
    _edge_loop(nj, t_sp, acc_sp, sidx_v, didx_v, rows_v, gsem, ssem)
    plsc.subcore_barrier()
    pltpu.sync_copy(acc_sp.at[sl], out_hbm.at[cid, sl])


@functools.partial(
    pl.kernel,
    mesh=_sc_mesh(),
    compiler_params=_SC_PARAMS,
    out_type=jax.ShapeDtypeStruct((2, NP, H), jnp.float32),
    scratch_types=[
        pltpu.VMEM_SHARED((NP, H), jnp.float32),
        pltpu.VMEM_SHARED((NP, H), jnp.float32),
        pltpu.VMEM((MAXJ, CHE), jnp.int32),
        pltpu.VMEM((MAXJ, CHE), jnp.int32),
        pltpu.VMEM((RING, CHE, H), jnp.float32),
        pltpu.VMEM((RPS, H), jnp.float32),
        pltpu.VMEM((RPS, H), jnp.float32),
        pltpu.VMEM((RPS, H), jnp.float32),
        pltpu.VMEM((RPS,), jnp.float32),
        pltpu.VMEM((RPS,), jnp.float32),
        pltpu.VMEM((RPS,), jnp.float32),
        pltpu.VMEM((H,), jnp.float32),
        pltpu.SemaphoreType.DMA((RING,)),
        pltpu.SemaphoreType.DMA((RING,)),
        pltpu.SemaphoreType.DMA,
    ],
)
def _edge_scatter_mid_sc(s1_hbm, degs_hbm, b1_hbm, edge_hbm, out_hbm,
                         acc_sp, t_sp, sidx_v, didx_v, rows_v,
                         s1a_v, s1b_v, t2_v, da_v, db_v, dinv_v, b1_v,
                         gsem, ssem, isem):
    cid = lax.axis_index("c")
    sid = lax.axis_index("s")
    wid = cid * 16 + sid
    start, nj = _tile_range(wid)
    sl = pl.ds(sid * RPS, RPS)
    loads = [
        pltpu.async_copy(edge_hbm.at[0, pl.ds(start, MAXJ)], sidx_v, isem),
        pltpu.async_copy(edge_hbm.at[1, pl.ds(start, MAXJ)], didx_v, isem),
        pltpu.async_copy(s1_hbm.at[0, sl], s1a_v, isem),
        pltpu.async_copy(s1_hbm.at[1, sl], s1b_v, isem),
        pltpu.async_copy(degs_hbm.at[0, sl], da_v, isem),
        pltpu.async_copy(degs_hbm.at[1, sl], db_v, isem),
        pltpu.async_copy(b1_hbm, b1_v, isem),
    ]
    for l in loads:
        l.wait()

    _compute_dinv(da_v, db_v, dinv_v)
    b1r = b1_v[...]

    @plsc.parallel_loop(0, RPS, unroll=8)
    def _mid(r):
        d = _splat(dinv_v, r)
        pre = d * (s1a_v[r] + s1b_v[r]) + b1r
        t2_v[r] = d * jnp.maximum(pre, 0.0)

    m2 = pltpu.async_copy(t2_v, t_sp.at[sl], isem)

    @pl.when(cid == 0)
    def _():
        pltpu.sync_copy(t2_v, acc_sp.at[sl])

    @pl.when(cid == 1)
    def _():
        for i in range(CHE):
            rows_v[0, i] = jnp.zeros((H,), jnp.float32)
        for k in range(RPS // CHE):
            pltpu.sync_copy(rows_v.at[0],
                            acc_sp.at[pl.ds(sid * RPS + k * CHE, CHE)])

    m2.wait()
    plsc.subcore_barrier()
    _edge_loop(nj, t_sp, acc_sp, sidx_v, didx_v, rows_v, gsem, ssem)
    plsc.subcore_barrier()
    pltpu.sync_copy(acc_sp.at[sl], out_hbm.at[cid, sl])


def _tc_mm_body(x_ref, w_ref, h_ref):
    h = jnp.dot(x_ref[...], w_ref[...], preferred_element_type=jnp.float32)
    h_ref[pl.ds(0, N), :] = h
    h_ref[pl.ds(N, NP - N), :] = jnp.zeros((NP - N, H), jnp.float32)


def _tc_out_body(s_ref, dsum_ref, w_ref, b_ref, o_ref):
    agg = lax.rsqrt(dsum_ref[...]) * (s_ref[0] + s_ref[1])
    z = jnp.dot(agg[:N], w_ref[...], preferred_element_type=jnp.float32) + b_ref[...]
    m = jnp.max(z, axis=1, keepdims=True)
    lse = m + jnp.log(jnp.sum(jnp.exp(z - m), axis=1, keepdims=True))
    o_ref[...] = z - lse


_tc_mm = pl.pallas_call(
    _tc_mm_body, out_shape=jax.ShapeDtypeStruct((NP, H), jnp.float32))
_tc_out = pl.pallas_call(
    _tc_out_body, out_shape=jax.ShapeDtypeStruct((N, C), jnp.float32))


def kernel(x, edge_index, W1, b1, W2, b2):
    edges = edge_index.astype(jnp.int32).reshape(2, NROW, CHE)

    h1 = _tc_mm(x, W1)
    degs = _deg_sc(edges)
    s1 = _edge_scale_scatter_sc(h1, degs, edges)
    s2 = _edge_scatter_mid_sc(s1, degs, b1, edges)
    dsum = (degs[0] + degs[1] + 1.0).reshape(NP, 1)
    return _tc_out(s2, dsum, W2, b2.reshape(1, C))
